# baseline (speedup 1.0000x reference)
import functools

import jax
import jax.numpy as jnp
from jax import lax
from jax.experimental import pallas as pl
from jax.experimental.pallas import tpu as pltpu
from jax.experimental.pallas import tpu_sc as plsc

B = 16384
D = 128
NC = 2
NS = 16
L = 16
NW = NC * NS
BPW = B // NW
CH = 128
NCH = BPW // CH
G = 16
NG = CH // G


CU = 16
NST = D // CU


def _sc_diff_kernel(user_hbm, pos_hbm, neg_hbm, utab_hbm, itab_hbm, out_hbm,
                    uidx_v, pidx_v, nidx_v,
                    u0_v, p0_v, n0_v, u1_v, p1_v, n1_v, scr_v, dot_v,
                    sem0, sem1):
    c = lax.axis_index("c")
    s = lax.axis_index("s")
    wid = s * NC + c

    pltpu.sync_copy(user_hbm.at[wid], uidx_v)
    pltpu.sync_copy(pos_hbm.at[wid], pidx_v)
    pltpu.sync_copy(neg_hbm.at[wid], nidx_v)

    lanes = lax.iota(jnp.int32, L)
    bufs = [(u0_v, p0_v, n0_v, sem0), (u1_v, p1_v, n1_v, sem1)]

    def issue(ch):
        u_b, p_b, n_b, sem = bufs[ch % 2]
        return (
            pltpu.async_copy(utab_hbm.at[uidx_v.at[ch]], u_b, sem),
            pltpu.async_copy(itab_hbm.at[pidx_v.at[ch]], p_b, sem),
            pltpu.async_copy(itab_hbm.at[nidx_v.at[ch]], n_b, sem),
        )

    diags = [lanes * L + ((lanes + l_) & (L - 1)) for l_ in range(L)]

    handles = {0: issue(0)}
    for ch in range(NCH):
        if ch + 1 < NCH:
            handles[ch + 1] = issue(ch + 1)
        for h in handles.pop(ch):
            h.wait()
        u_b, p_b, n_b, _ = bufs[ch % 2]

        def group_body(g, carry, ch=ch, u_b=u_b, p_b=p_b, n_b=n_b):
            for r in range(G):
                acc = None
                for j in range(D // L):
                    uu = u_b[g * G + r, pl.ds(j * L, L)]
                    pp = p_b[g * G + r, pl.ds(j * L, L)]
                    nn = n_b[g * G + r, pl.ds(j * L, L)]
                    t = uu * (pp - nn)
                    acc = t if acc is None else acc + t
                scr_v[pl.ds(r * L, L)] = acc
            terms = [plsc.load_gather(scr_v, [d]) for d in diags]
            while len(terms) > 1:
                terms = [a + b for a, b in zip(terms[::2], terms[1::2])]
            dot_v[pl.ds(ch * CH + g * G, G)] = terms[0]
            return carry

        lax.fori_loop(0, NG, group_body, 0)

    pltpu.sync_copy(dot_v, out_hbm.at[wid])


def _diff_on_sc(user, pos_item, neg_item, user_table, item_table):
    mesh = plsc.VectorSubcoreMesh(core_axis_name="c", subcore_axis_name="s")
    kfn = pl.kernel(
        _sc_diff_kernel,
        mesh=mesh,
        compiler_params=pltpu.CompilerParams(needs_layout_passes=False),
        out_type=jax.ShapeDtypeStruct((NW, BPW), jnp.float32),
        scratch_types=[
            pltpu.VMEM((NCH, CH), jnp.int32),
            pltpu.VMEM((NCH, CH), jnp.int32),
            pltpu.VMEM((NCH, CH), jnp.int32),
            pltpu.VMEM((CH, D), jnp.float32),
            pltpu.VMEM((CH, D), jnp.float32),
            pltpu.VMEM((CH, D), jnp.float32),
            pltpu.VMEM((CH, D), jnp.float32),
            pltpu.VMEM((CH, D), jnp.float32),
            pltpu.VMEM((CH, D), jnp.float32),
            pltpu.VMEM((G * L,), jnp.float32),
            pltpu.VMEM((BPW,), jnp.float32),
            pltpu.SemaphoreType.DMA,
            pltpu.SemaphoreType.DMA,
        ],
    )
    diff = kfn(
        user.reshape(NW, NCH, CH),
        pos_item.reshape(NW, NCH, CH),
        neg_item.reshape(NW, NCH, CH),
        user_table,
        item_table,
    )
    return diff.reshape(B)


def _loss_body(x_ref, o_ref):
    x = x_ref[...]
    t = -jnp.log(1e-8 + jax.nn.sigmoid(x))
    o_ref[0, 0] = jnp.sum(t) * (1.0 / B)


def _loss_on_tc(diff):
    out = pl.pallas_call(
        _loss_body,
        out_shape=jax.ShapeDtypeStruct((1, 1), jnp.float32),
        out_specs=pl.BlockSpec(memory_space=pltpu.SMEM),
    )(diff.reshape(B // D, D))
    return out[0, 0]


@jax.jit
def kernel(user, pos_item, neg_item, user_table, item_table):
    diff = _diff_on_sc(user, pos_item, neg_item, user_table, item_table)
    return _loss_on_tc(diff)


# speedup vs baseline: 1.0601x; 1.0601x over previous
import jax
import jax.numpy as jnp
from jax import lax
from jax.experimental import pallas as pl
from jax.experimental.pallas import tpu as pltpu
from jax.experimental.pallas import tpu_sc as plsc

B = 16384
D = 128
NC = 2
NS = 16
L = 16
NW = NC * NS
BPW = B // NW
CH = 128
NCH = BPW // CH
G = 16
NG = CH // G
RPW = BPW // D


def _sc_diff_kernel(user_hbm, pos_hbm, neg_hbm, utab_hbm, itab_hbm, out_hbm,
                    uidx_v, pidx_v, nidx_v,
                    u0_v, p0_v, n0_v, u1_v, p1_v, n1_v, scr_v, dot_v,
                    isem, sem0, sem1):
    c = lax.axis_index("c")
    s = lax.axis_index("s")
    wid = s * NC + c
    base = wid * BPW

    ci = pltpu.async_copy(user_hbm.at[pl.ds(base, BPW)], uidx_v, isem)
    cp = pltpu.async_copy(pos_hbm.at[pl.ds(base, BPW)], pidx_v, isem)
    cn = pltpu.async_copy(neg_hbm.at[pl.ds(base, BPW)], nidx_v, isem)
    ci.wait()
    cp.wait()
    cn.wait()

    lanes = lax.iota(jnp.int32, L)
    bufs = [(u0_v, p0_v, n0_v, sem0), (u1_v, p1_v, n1_v, sem1)]

    def issue(ch):
        u_b, p_b, n_b, sem = bufs[ch % 2]
        sl = pl.ds(ch * CH, CH)
        return (
            pltpu.async_copy(utab_hbm.at[uidx_v.at[sl]], u_b, sem),
            pltpu.async_copy(itab_hbm.at[pidx_v.at[sl]], p_b, sem),
            pltpu.async_copy(itab_hbm.at[nidx_v.at[sl]], n_b, sem),
        )

    diags = [lanes * L + ((lanes + l_) & (L - 1)) for l_ in range(L)]

    handles = {0: issue(0)}
    for ch in range(NCH):
        if ch + 1 < NCH:
            handles[ch + 1] = issue(ch + 1)
        for h in handles.pop(ch):
            h.wait()
        u_b, p_b, n_b, _ = bufs[ch % 2]

        def group_body(g, carry, ch=ch, u_b=u_b, p_b=p_b, n_b=n_b):
            for r in range(G):
                acc = None
                for j in range(D // L):
                    uu = u_b[g * G + r, pl.ds(j * L, L)]
                    pp = p_b[g * G + r, pl.ds(j * L, L)]
                    nn = n_b[g * G + r, pl.ds(j * L, L)]
                    t = uu * (pp - nn)
                    acc = t if acc is None else acc + t
                scr_v[pl.ds(r * L, L)] = acc
            terms = [plsc.load_gather(scr_v, [d]) for d in diags]
            while len(terms) > 1:
                terms = [a + b for a, b in zip(terms[::2], terms[1::2])]
            dot_v[ch, pl.ds(g * G, G)] = terms[0]
            return carry

        lax.fori_loop(0, NG, group_body, 0)

    pltpu.sync_copy(dot_v, out_hbm.at[pl.ds(wid * RPW, RPW)])


def _diff_on_sc(user, pos_item, neg_item, user_table, item_table):
    mesh = plsc.VectorSubcoreMesh(core_axis_name="c", subcore_axis_name="s")
    kfn = pl.kernel(
        _sc_diff_kernel,
        mesh=mesh,
        compiler_params=pltpu.CompilerParams(needs_layout_passes=False),
        out_type=jax.ShapeDtypeStruct((B // D, D), jnp.float32),
        scratch_types=[
            pltpu.VMEM((BPW,), jnp.int32),
            pltpu.VMEM((BPW,), jnp.int32),
            pltpu.VMEM((BPW,), jnp.int32),
            pltpu.VMEM((CH, D), jnp.float32),
            pltpu.VMEM((CH, D), jnp.float32),
            pltpu.VMEM((CH, D), jnp.float32),
            pltpu.VMEM((CH, D), jnp.float32),
            pltpu.VMEM((CH, D), jnp.float32),
            pltpu.VMEM((CH, D), jnp.float32),
            pltpu.VMEM((G * L,), jnp.float32),
            pltpu.VMEM((NCH, CH), jnp.float32),
            pltpu.SemaphoreType.DMA,
            pltpu.SemaphoreType.DMA,
            pltpu.SemaphoreType.DMA,
        ],
    )
    return kfn(user, pos_item, neg_item, user_table, item_table)


def _loss_body(x_ref, o_ref):
    x = x_ref[...]
    t = -jnp.log(1e-8 + jax.nn.sigmoid(x))
    o_ref[0, 0] = jnp.sum(t) * (1.0 / B)


def _loss_on_tc(diff):
    out = pl.pallas_call(
        _loss_body,
        out_shape=jax.ShapeDtypeStruct((1, 1), jnp.float32),
        out_specs=pl.BlockSpec(memory_space=pltpu.SMEM),
    )(diff)
    return out[0, 0]


@jax.jit
def kernel(user, pos_item, neg_item, user_table, item_table):
    diff = _diff_on_sc(user.astype(jnp.int32), pos_item.astype(jnp.int32),
                       neg_item.astype(jnp.int32), user_table, item_table)
    return _loss_on_tc(diff)
